# baseline (speedup 1.0000x reference)
import numpy as np
import jax
import jax.numpy as jnp
from jax.experimental import pallas as pl
from jax.experimental.pallas import tpu as pltpu

_N_ROWS = 16384
_N_COLS = 2048
_CHUNK_R = 512
_N_CHUNKS = _N_ROWS // _CHUNK_R
_N_BUF = 8


def _uniform_consts() -> np.ndarray:
    ks0, ks1 = np.uint32(0), np.uint32(42)
    ks2 = np.uint32(ks0 ^ ks1 ^ np.uint32(0x1BD11BDA))
    ks = [ks0, ks1, ks2]
    rot = [(13, 15, 26, 6), (17, 29, 16, 24)]

    def rotl(x, r):
        r = np.uint32(r)
        return ((x << r) | (x >> np.uint32(32 - r))).astype(np.uint32)

    x0 = np.full(_N_COLS, ks0, dtype=np.uint32)
    x1 = (np.arange(_N_COLS, dtype=np.uint32) + ks1).astype(np.uint32)
    for i in range(5):
        for r in rot[i % 2]:
            x0 = (x0 + x1).astype(np.uint32)
            x1 = rotl(x1, r)
            x1 = (x1 ^ x0).astype(np.uint32)
        x0 = (x0 + ks[(i + 1) % 3]).astype(np.uint32)
        x1 = (x1 + ks[(i + 2) % 3] + np.uint32(i + 1)).astype(np.uint32)
    bits = (x0 ^ x1).astype(np.uint32)
    fb = ((bits >> np.uint32(9)) | np.uint32(0x3F800000)).astype(np.uint32)
    u = fb.view(np.float32) - np.float32(1.0)
    return np.broadcast_to(u, (8, _N_COLS)).copy()


_U_TILE = _uniform_consts()


def _ring_body(idx_ref, prob_ref, u_ref, in_hbm, out_hbm, buf, in_sems, out_sems):
    idx = idx_ref[0]

    def in_copy(c, b):
        return pltpu.make_async_copy(
            in_hbm.at[pl.ds(c * _CHUNK_R, _CHUNK_R), :], buf.at[b],
            in_sems.at[b])

    def out_copy(c, b):
        return pltpu.make_async_copy(
            buf.at[b], out_hbm.at[pl.ds(c * _CHUNK_R, _CHUNK_R), :],
            out_sems.at[b])

    for c in range(_N_BUF):
        in_copy(c, c).start()

    for c in range(_N_CHUNKS):
        b = c % _N_BUF
        in_copy(c, b).wait()

        @pl.when(c == idx // _CHUNK_R)
        def _flip_row():
            r = idx % _CHUNK_R
            row = buf[b, pl.ds(r, 1), :]
            sample = (u_ref[pl.ds(0, 1), :] < prob_ref[0]).astype(jnp.float32)
            buf[b, pl.ds(r, 1), :] = jnp.abs(row - sample)

        out_copy(c, b).start()
        nxt = c + _N_BUF
        if nxt < _N_CHUNKS:
            out_copy(c, b).wait()
            in_copy(nxt, b).start()
    for c in range(_N_CHUNKS - _N_BUF, _N_CHUNKS):
        out_copy(c, c % _N_BUF).wait()


def kernel(tensor, tensor_slice_index, probability):
    idx = jnp.asarray(tensor_slice_index, jnp.int32).reshape((1,))
    prob = jnp.asarray(probability, jnp.float32).reshape((1,))
    u = jnp.asarray(_U_TILE)
    out = pl.pallas_call(
        _ring_body,
        in_specs=[
            pl.BlockSpec(memory_space=pltpu.SMEM),
            pl.BlockSpec(memory_space=pltpu.SMEM),
            pl.BlockSpec(memory_space=pltpu.VMEM),
            pl.BlockSpec(memory_space=pltpu.MemorySpace.HBM),
        ],
        out_specs=pl.BlockSpec(memory_space=pltpu.MemorySpace.HBM),
        out_shape=jax.ShapeDtypeStruct((_N_ROWS, _N_COLS), jnp.float32),
        scratch_shapes=[
            pltpu.VMEM((_N_BUF, _CHUNK_R, _N_COLS), jnp.float32),
            pltpu.SemaphoreType.DMA((_N_BUF,)),
            pltpu.SemaphoreType.DMA((_N_BUF,)),
        ],
    )(idx, prob, u, tensor)
    return (out, tensor_slice_index)


# speedup vs baseline: 1.0108x; 1.0108x over previous
import numpy as np
import jax
import jax.numpy as jnp
from jax.experimental import pallas as pl
from jax.experimental.pallas import tpu as pltpu

_N_ROWS = 16384
_N_COLS = 2048
_CHUNK_R = 1024
_N_CHUNKS = _N_ROWS // _CHUNK_R
_N_BUF = 6


def _uniform_consts() -> np.ndarray:
    ks0, ks1 = np.uint32(0), np.uint32(42)
    ks2 = np.uint32(ks0 ^ ks1 ^ np.uint32(0x1BD11BDA))
    ks = [ks0, ks1, ks2]
    rot = [(13, 15, 26, 6), (17, 29, 16, 24)]

    def rotl(x, r):
        r = np.uint32(r)
        return ((x << r) | (x >> np.uint32(32 - r))).astype(np.uint32)

    x0 = np.full(_N_COLS, ks0, dtype=np.uint32)
    x1 = (np.arange(_N_COLS, dtype=np.uint32) + ks1).astype(np.uint32)
    for i in range(5):
        for r in rot[i % 2]:
            x0 = (x0 + x1).astype(np.uint32)
            x1 = rotl(x1, r)
            x1 = (x1 ^ x0).astype(np.uint32)
        x0 = (x0 + ks[(i + 1) % 3]).astype(np.uint32)
        x1 = (x1 + ks[(i + 2) % 3] + np.uint32(i + 1)).astype(np.uint32)
    bits = (x0 ^ x1).astype(np.uint32)
    fb = ((bits >> np.uint32(9)) | np.uint32(0x3F800000)).astype(np.uint32)
    u = fb.view(np.float32) - np.float32(1.0)
    return np.broadcast_to(u, (8, _N_COLS)).copy()


_U_TILE = _uniform_consts()


def _ring_body(idx_ref, prob_ref, u_ref, in_hbm, out_hbm, buf, in_sems, out_sems):
    idx = idx_ref[0]

    def in_copy(c, b):
        return pltpu.make_async_copy(
            in_hbm.at[pl.ds(c * _CHUNK_R, _CHUNK_R), :], buf.at[b],
            in_sems.at[b])

    def out_copy(c, b):
        return pltpu.make_async_copy(
            buf.at[b], out_hbm.at[pl.ds(c * _CHUNK_R, _CHUNK_R), :],
            out_sems.at[b])

    for c in range(_N_BUF):
        in_copy(c, c).start()

    for c in range(_N_CHUNKS):
        b = c % _N_BUF
        in_copy(c, b).wait()

        @pl.when(c == idx // _CHUNK_R)
        def _flip_row():
            r = idx % _CHUNK_R
            row = buf[b, pl.ds(r, 1), :]
            sample = (u_ref[pl.ds(0, 1), :] < prob_ref[0]).astype(jnp.float32)
            buf[b, pl.ds(r, 1), :] = jnp.abs(row - sample)

        out_copy(c, b).start()
        nxt = c + _N_BUF
        if nxt < _N_CHUNKS:
            out_copy(c, b).wait()
            in_copy(nxt, b).start()
    for c in range(_N_CHUNKS - _N_BUF, _N_CHUNKS):
        out_copy(c, c % _N_BUF).wait()


def kernel(tensor, tensor_slice_index, probability):
    idx = jnp.asarray(tensor_slice_index, jnp.int32).reshape((1,))
    prob = jnp.asarray(probability, jnp.float32).reshape((1,))
    u = jnp.asarray(_U_TILE)
    out = pl.pallas_call(
        _ring_body,
        in_specs=[
            pl.BlockSpec(memory_space=pltpu.SMEM),
            pl.BlockSpec(memory_space=pltpu.SMEM),
            pl.BlockSpec(memory_space=pltpu.VMEM),
            pl.BlockSpec(memory_space=pltpu.MemorySpace.HBM),
        ],
        out_specs=pl.BlockSpec(memory_space=pltpu.MemorySpace.HBM),
        out_shape=jax.ShapeDtypeStruct((_N_ROWS, _N_COLS), jnp.float32),
        scratch_shapes=[
            pltpu.VMEM((_N_BUF, _CHUNK_R, _N_COLS), jnp.float32),
            pltpu.SemaphoreType.DMA((_N_BUF,)),
            pltpu.SemaphoreType.DMA((_N_BUF,)),
        ],
    )(idx, prob, u, tensor)
    return (out, tensor_slice_index)


# speedup vs baseline: 1.0129x; 1.0020x over previous
import numpy as np
import jax
import jax.numpy as jnp
from jax.experimental import pallas as pl
from jax.experimental.pallas import tpu as pltpu

_N_ROWS = 16384
_N_COLS = 2048
_CHUNK_R = 2048
_N_CHUNKS = _N_ROWS // _CHUNK_R
_N_BUF = 3


def _uniform_consts() -> np.ndarray:
    ks0, ks1 = np.uint32(0), np.uint32(42)
    ks2 = np.uint32(ks0 ^ ks1 ^ np.uint32(0x1BD11BDA))
    ks = [ks0, ks1, ks2]
    rot = [(13, 15, 26, 6), (17, 29, 16, 24)]

    def rotl(x, r):
        r = np.uint32(r)
        return ((x << r) | (x >> np.uint32(32 - r))).astype(np.uint32)

    x0 = np.full(_N_COLS, ks0, dtype=np.uint32)
    x1 = (np.arange(_N_COLS, dtype=np.uint32) + ks1).astype(np.uint32)
    for i in range(5):
        for r in rot[i % 2]:
            x0 = (x0 + x1).astype(np.uint32)
            x1 = rotl(x1, r)
            x1 = (x1 ^ x0).astype(np.uint32)
        x0 = (x0 + ks[(i + 1) % 3]).astype(np.uint32)
        x1 = (x1 + ks[(i + 2) % 3] + np.uint32(i + 1)).astype(np.uint32)
    bits = (x0 ^ x1).astype(np.uint32)
    fb = ((bits >> np.uint32(9)) | np.uint32(0x3F800000)).astype(np.uint32)
    u = fb.view(np.float32) - np.float32(1.0)
    return np.broadcast_to(u, (8, _N_COLS)).copy()


_U_TILE = _uniform_consts()


def _ring_body(idx_ref, prob_ref, u_ref, in_hbm, out_hbm, buf, in_sems, out_sems):
    idx = idx_ref[0]

    def in_copy(c, b):
        return pltpu.make_async_copy(
            in_hbm.at[pl.ds(c * _CHUNK_R, _CHUNK_R), :], buf.at[b],
            in_sems.at[b])

    def out_copy(c, b):
        return pltpu.make_async_copy(
            buf.at[b], out_hbm.at[pl.ds(c * _CHUNK_R, _CHUNK_R), :],
            out_sems.at[b])

    for c in range(_N_BUF):
        in_copy(c, c).start()

    for c in range(_N_CHUNKS):
        b = c % _N_BUF
        in_copy(c, b).wait()

        @pl.when(c == idx // _CHUNK_R)
        def _flip_row():
            r = idx % _CHUNK_R
            row = buf[b, pl.ds(r, 1), :]
            sample = (u_ref[pl.ds(0, 1), :] < prob_ref[0]).astype(jnp.float32)
            buf[b, pl.ds(r, 1), :] = jnp.abs(row - sample)

        out_copy(c, b).start()
        nxt = c + _N_BUF
        if nxt < _N_CHUNKS:
            out_copy(c, b).wait()
            in_copy(nxt, b).start()
    for c in range(_N_CHUNKS - _N_BUF, _N_CHUNKS):
        out_copy(c, c % _N_BUF).wait()


def kernel(tensor, tensor_slice_index, probability):
    idx = jnp.asarray(tensor_slice_index, jnp.int32).reshape((1,))
    prob = jnp.asarray(probability, jnp.float32).reshape((1,))
    u = jnp.asarray(_U_TILE)
    out = pl.pallas_call(
        _ring_body,
        in_specs=[
            pl.BlockSpec(memory_space=pltpu.SMEM),
            pl.BlockSpec(memory_space=pltpu.SMEM),
            pl.BlockSpec(memory_space=pltpu.VMEM),
            pl.BlockSpec(memory_space=pltpu.MemorySpace.HBM),
        ],
        out_specs=pl.BlockSpec(memory_space=pltpu.MemorySpace.HBM),
        out_shape=jax.ShapeDtypeStruct((_N_ROWS, _N_COLS), jnp.float32),
        scratch_shapes=[
            pltpu.VMEM((_N_BUF, _CHUNK_R, _N_COLS), jnp.float32),
            pltpu.SemaphoreType.DMA((_N_BUF,)),
            pltpu.SemaphoreType.DMA((_N_BUF,)),
        ],
    )(idx, prob, u, tensor)
    return (out, tensor_slice_index)
